# baseline (speedup 1.0000x reference)
import jax
import jax.numpy as jnp
from jax import lax
from jax.experimental import pallas as pl
from jax.experimental.pallas import tpu as pltpu
from jax.experimental.pallas import tpu_sc as plsc

B = 16384
D_FACE = 64
D_BODY = 74
D_ID = D_BODY - 10
NC, NS = 2, 16
NW = NC * NS
BPW = B // NW
CHUNK = 128
NCH = BPW // CHUNK
UNROLL = 8


def _sc_gather(face_idx, body_idx, face_tab, body_tab,
               out_face, out_body, out_shape,
               idx_f, idx_b, rows_f, rows_b, sem_f, sem_b):
    wid = lax.axis_index("s") * NC + lax.axis_index("c")
    base = wid * BPW
    pltpu.sync_copy(face_idx.at[pl.ds(wid * NCH, NCH)], idx_f)
    pltpu.sync_copy(body_idx.at[pl.ds(wid, 1)], idx_b)
    f_copies = [
        pltpu.async_copy(face_tab.at[idx_f.at[j]],
                         rows_f.at[pl.ds(j * CHUNK, CHUNK)], sem_f)
        for j in range(NCH)
    ]
    def issue(r0, _):
        vec = idx_b[0, pl.ds(r0 * 16, 16)]
        for u in range(16):
            pltpu.async_copy(body_tab.at[pl.ds(vec[u], 1)],
                             rows_b.at[pl.ds(r0 * 16 + u, 1)], sem_b)
        return ()
    lax.fori_loop(0, BPW // 16, issue, (), unroll=False)
    pltpu.make_async_copy(body_tab.at[pl.ds(0, BPW)], rows_b, sem_b).wait()
    for c in f_copies:
        c.wait()
    pltpu.sync_copy(rows_f, out_face.at[pl.ds(base, BPW)])
    pltpu.sync_copy(rows_b.at[pl.ds(0, BPW), pl.ds(0, D_ID)],
                    out_body.at[pl.ds(base, BPW)])
    pltpu.sync_copy(rows_b.at[pl.ds(0, BPW), pl.ds(D_ID, 10)],
                    out_shape.at[pl.ds(base, BPW)])


def kernel(rand_id_face, rand_id_body, id_face_sampler, id_shape_sampler_body):
    mesh = plsc.VectorSubcoreMesh(core_axis_name="c", subcore_axis_name="s")
    f = pl.kernel(
        _sc_gather,
        mesh=mesh,
        compiler_params=pltpu.CompilerParams(use_tc_tiling_on_sc=False),
        out_type=(
            jax.ShapeDtypeStruct((B, D_FACE), jnp.float32),
            jax.ShapeDtypeStruct((B, D_ID), jnp.float32),
            jax.ShapeDtypeStruct((B, 10), jnp.float32),
        ),
        scratch_types=[
            pltpu.VMEM((NCH, CHUNK), jnp.int32),
            pltpu.VMEM((1, BPW), jnp.int32),
            pltpu.VMEM((BPW, D_FACE), jnp.float32),
            pltpu.VMEM((BPW, D_BODY), jnp.float32),
            pltpu.SemaphoreType.DMA,
            pltpu.SemaphoreType.DMA,
        ],
    )
    return f(rand_id_face.reshape(NW * NCH, CHUNK),
             rand_id_body.reshape(NW, BPW),
             id_face_sampler, id_shape_sampler_body)


# speedup vs baseline: 2.5506x; 2.5506x over previous
import jax
import jax.numpy as jnp
from jax import lax
from jax.experimental import pallas as pl
from jax.experimental.pallas import tpu as pltpu

B = 16384
D_FACE = 64
D_BODY = 74
D_ID = D_BODY - 10
UNROLL = 8


def _gather_kernel(idx_f, idx_b, face_hbm, body_hbm,
                   out_f, out_b, out_s, rows_b, sem_f, sem_b):
    def issue(j0, _):
        for u in range(UNROLL):
            j = j0 * UNROLL + u
            pltpu.make_async_copy(
                face_hbm.at[pl.ds(idx_f[j], 1)],
                out_f.at[pl.ds(j, 1)], sem_f).start()
            pltpu.make_async_copy(
                body_hbm.at[pl.ds(idx_b[j], 1)],
                rows_b.at[pl.ds(j, 1)], sem_b).start()
        return ()
    lax.fori_loop(0, B // UNROLL, issue, ())

    def drain(j0, _):
        for u in range(UNROLL):
            j = j0 * UNROLL + u
            pltpu.make_async_copy(
                face_hbm.at[pl.ds(0, 1)],
                out_f.at[pl.ds(j, 1)], sem_f).wait()
            pltpu.make_async_copy(
                body_hbm.at[pl.ds(0, 1)],
                rows_b.at[pl.ds(j, 1)], sem_b).wait()
        return ()
    lax.fori_loop(0, B // UNROLL, drain, ())

    out_b[...] = rows_b[:, :D_ID]
    out_s[...] = rows_b[:, D_ID:D_BODY]


def kernel(rand_id_face, rand_id_body, id_face_sampler, id_shape_sampler_body):
    grid_spec = pltpu.PrefetchScalarGridSpec(
        num_scalar_prefetch=2,
        in_specs=[
            pl.BlockSpec(memory_space=pltpu.MemorySpace.HBM),
            pl.BlockSpec(memory_space=pltpu.MemorySpace.HBM),
        ],
        scratch_shapes=[
            pltpu.VMEM((B, D_BODY), jnp.float32),
            pltpu.SemaphoreType.DMA,
            pltpu.SemaphoreType.DMA,
        ],
    )
    f = pl.pallas_call(
        _gather_kernel,
        grid_spec=grid_spec,
        out_shape=(
            jax.ShapeDtypeStruct((B, D_FACE), jnp.float32),
            jax.ShapeDtypeStruct((B, D_ID), jnp.float32),
            jax.ShapeDtypeStruct((B, 10), jnp.float32),
        ),
    )
    return f(rand_id_face, rand_id_body, id_face_sampler, id_shape_sampler_body)


# speedup vs baseline: 2.6913x; 1.0552x over previous
import jax
import jax.numpy as jnp
from jax import lax
from jax.experimental import pallas as pl
from jax.experimental.pallas import tpu as pltpu

B = 16384
D_FACE = 64
D_BODY = 74
D_ID = D_BODY - 10
UNROLL = 8


def _gather_kernel(idx_f, idx_b, face_hbm, body_hbm,
                   out_f, out_b, out_s, rows_b, sem_f, sem_b):
    def issue(j0, _):
        for u in range(UNROLL):
            j = j0 * UNROLL + u
            pltpu.make_async_copy(
                face_hbm.at[pl.ds(idx_f[j], 1)],
                out_f.at[pl.ds(j, 1)], sem_f).start()
            pltpu.make_async_copy(
                body_hbm.at[pl.ds(idx_b[j], 1)],
                rows_b.at[pl.ds(j, 1)], sem_b).start()
        return ()
    lax.fori_loop(0, B // UNROLL, issue, ())

    pltpu.make_async_copy(face_hbm.at[pl.ds(0, B)], out_f, sem_f).wait()
    pltpu.make_async_copy(body_hbm.at[pl.ds(0, B)], rows_b, sem_b).wait()

    out_b[...] = rows_b[:, :D_ID]
    out_s[...] = rows_b[:, D_ID:D_BODY]


def kernel(rand_id_face, rand_id_body, id_face_sampler, id_shape_sampler_body):
    grid_spec = pltpu.PrefetchScalarGridSpec(
        num_scalar_prefetch=2,
        in_specs=[
            pl.BlockSpec(memory_space=pltpu.MemorySpace.HBM),
            pl.BlockSpec(memory_space=pltpu.MemorySpace.HBM),
        ],
        scratch_shapes=[
            pltpu.VMEM((B, D_BODY), jnp.float32),
            pltpu.SemaphoreType.DMA,
            pltpu.SemaphoreType.DMA,
        ],
    )
    f = pl.pallas_call(
        _gather_kernel,
        grid_spec=grid_spec,
        out_shape=(
            jax.ShapeDtypeStruct((B, D_FACE), jnp.float32),
            jax.ShapeDtypeStruct((B, D_ID), jnp.float32),
            jax.ShapeDtypeStruct((B, 10), jnp.float32),
        ),
    )
    return f(rand_id_face, rand_id_body, id_face_sampler, id_shape_sampler_body)


# speedup vs baseline: 2.6967x; 1.0020x over previous
import jax
import jax.numpy as jnp
from jax import lax
from jax.experimental import pallas as pl
from jax.experimental.pallas import tpu as pltpu

B = 16384
D_FACE = 64
D_BODY = 74
D_ID = D_BODY - 10
UNROLL = 8
NSEM = 4
SPAN = B // NSEM


def _gather_kernel(idx_f, idx_b, face_hbm, body_hbm,
                   out_f, out_b, out_s, rows_b, sems_f, sems_b):
    def issue(j0, _):
        for u in range(UNROLL):
            j = j0 * UNROLL + u
            pltpu.make_async_copy(
                face_hbm.at[pl.ds(idx_f[j], 1)],
                out_f.at[pl.ds(j, 1)], sems_f.at[u % NSEM]).start()
            pltpu.make_async_copy(
                body_hbm.at[pl.ds(idx_b[j], 1)],
                rows_b.at[pl.ds(j, 1)], sems_b.at[u % NSEM]).start()
        return ()
    lax.fori_loop(0, B // UNROLL, issue, ())

    for s in range(NSEM):
        pltpu.make_async_copy(face_hbm.at[pl.ds(0, SPAN)],
                              out_f.at[pl.ds(0, SPAN)], sems_f.at[s]).wait()
        pltpu.make_async_copy(body_hbm.at[pl.ds(0, SPAN)],
                              rows_b.at[pl.ds(0, SPAN)], sems_b.at[s]).wait()

    out_b[...] = rows_b[:, :D_ID]
    out_s[...] = rows_b[:, D_ID:D_BODY]


def kernel(rand_id_face, rand_id_body, id_face_sampler, id_shape_sampler_body):
    grid_spec = pltpu.PrefetchScalarGridSpec(
        num_scalar_prefetch=2,
        in_specs=[
            pl.BlockSpec(memory_space=pltpu.MemorySpace.HBM),
            pl.BlockSpec(memory_space=pltpu.MemorySpace.HBM),
        ],
        scratch_shapes=[
            pltpu.VMEM((B, D_BODY), jnp.float32),
            pltpu.SemaphoreType.DMA((NSEM,)),
            pltpu.SemaphoreType.DMA((NSEM,)),
        ],
    )
    f = pl.pallas_call(
        _gather_kernel,
        grid_spec=grid_spec,
        out_shape=(
            jax.ShapeDtypeStruct((B, D_FACE), jnp.float32),
            jax.ShapeDtypeStruct((B, D_ID), jnp.float32),
            jax.ShapeDtypeStruct((B, 10), jnp.float32),
        ),
    )
    return f(rand_id_face, rand_id_body, id_face_sampler, id_shape_sampler_body)
